# baseline (speedup 1.0000x reference)
import jax
import jax.numpy as jnp
from jax import lax
from jax.experimental import pallas as pl
from jax.experimental.pallas import tpu as pltpu
from jax.experimental.pallas import tpu_sc as plsc

NUM_KEYS = 1000000
EMBED_DIM = 32
N_VOX = 262144

NC = 2
NS = 16
L = 16
NW = NC * NS
VPW = N_VOX // NW
C = 128
ROWS = C * 8
OROWS = C * EMBED_DIM // 128
NCHUNK = VPW // C


def _body(table_hbm, feats_hbm, p_hbm, out_hbm,
          idxs_v, idx1d_v, rows_v, p_v, out_v,
          gsem0, gsem1, ssem0, ssem1, osem0, osem1):
    cid = lax.axis_index("c")
    sid = lax.axis_index("s")
    wid = sid * NC + cid
    gsems = (gsem0, gsem1)
    ssems = (ssem0, ssem1)
    osems = (osem0, osem1)

    def stage(g, b):
        vbase = wid * VPW + g * C
        pltpu.async_copy(feats_hbm.at[pl.ds(vbase, C)], idxs_v.at[b],
                         ssems[b])
        pltpu.async_copy(p_hbm.at[pl.ds(vbase, C)], p_v.at[b], ssems[b])

    def wait_stage(b):
        pltpu.make_async_copy(feats_hbm.at[pl.ds(0, C)], idxs_v.at[b],
                              ssems[b]).wait()
        pltpu.make_async_copy(p_hbm.at[pl.ds(0, C)], p_v.at[b],
                              ssems[b]).wait()

    def transpose_idx(b):
        def tr(i, c2):
            lanes = lax.iota(jnp.int32, L) + i * L
            for j in range(8):
                v = plsc.load_gather(idxs_v.at[b],
                                     [lanes, jnp.full((L,), j, jnp.int32)])
                idx1d_v[b, j, pl.ds(i * L, L)] = v
            return c2

        lax.fori_loop(0, C // L, tr, 0)

    def fire(g, b):
        for j in range(8):
            pltpu.async_copy(table_hbm.at[idx1d_v.at[b].at[j]],
                             rows_v.at[b].at[pl.ds(j * C, C)], gsems[b])

    def drain_gathers(b):
        pltpu.make_async_copy(table_hbm.at[pl.ds(0, ROWS)],
                              rows_v.at[b], gsems[b]).wait()

    def flush_out(g, b):
        vbase = wid * VPW + g * C
        pltpu.async_copy(out_v.at[b], out_hbm.at[pl.ds(vbase, C)],
                         osems[b])

    def wait_out(b):
        pltpu.make_async_copy(out_hbm.at[pl.ds(0, C)], out_v.at[b],
                              osems[b]).wait()

    def compute(g, b):
        def group(i, c2):
            vb = i * L
            lanes = lax.iota(jnp.int32, L) + vb
            px = plsc.load_gather(p_v.at[b],
                                  [lanes, jnp.full((L,), 0, jnp.int32)])
            py = plsc.load_gather(p_v.at[b],
                                  [lanes, jnp.full((L,), 1, jnp.int32)])
            pz = plsc.load_gather(p_v.at[b],
                                  [lanes, jnp.full((L,), 2, jnp.int32)])
            qx = 1.0 - px
            qy = 1.0 - py
            qz = 1.0 - pz
            w = []
            for j in range(8):
                wx = px if (j >> 2) & 1 else qx
                wy = py if (j >> 1) & 1 else qy
                wz = pz if j & 1 else qz
                w.append(wx * wy * wz)
            for lane in range(L):
                for h in range(2):
                    t = [w[j][lane] *
                         rows_v[b, j * C + vb + lane, pl.ds(h * L, L)]
                         for j in range(8)]
                    acc = ((t[0] + t[1]) + (t[2] + t[3])) + \
                          ((t[4] + t[5]) + (t[6] + t[7]))
                    out_v[b, vb + lane, pl.ds(h * L, L)] = acc
            return c2

        lax.fori_loop(0, C // L, group, 0)

    stage(0, 0)
    wait_stage(0)
    transpose_idx(0)
    fire(0, 0)
    stage(1, 1)

    def pair(t, carry):
        for bb in range(2):
            g = 2 * t + bb
            b, nb = bb, 1 - bb
            drain_gathers(b)

            @pl.when(g + 1 < NCHUNK)
            def _():
                wait_stage(nb)
                transpose_idx(nb)
                fire(g + 1, nb)

            @pl.when(g >= 2)
            def _():
                wait_out(b)

            compute(g, b)

            @pl.when(g + 2 < NCHUNK)
            def _():
                stage(g + 2, b)

            flush_out(g, b)
        return carry

    lax.fori_loop(0, NCHUNK // 2, pair, 0)
    wait_out(0)
    wait_out(1)


@jax.jit
def _sve(table, feats, p):
    mesh = plsc.VectorSubcoreMesh(core_axis_name="c", subcore_axis_name="s",
                                  num_cores=NC, num_subcores=NS)
    f = pl.kernel(
        _body,
        out_type=jax.ShapeDtypeStruct((N_VOX, EMBED_DIM), jnp.float32),
        mesh=mesh,
        scratch_types=[
            pltpu.VMEM((2, C, 8), jnp.int32),
            pltpu.VMEM((2, 8, C), jnp.int32),
            pltpu.VMEM((2, ROWS, EMBED_DIM), jnp.float32),
            pltpu.VMEM((2, C, 3), jnp.float32),
            pltpu.VMEM((2, C, EMBED_DIM), jnp.float32),
            pltpu.SemaphoreType.DMA,
            pltpu.SemaphoreType.DMA,
            pltpu.SemaphoreType.DMA,
            pltpu.SemaphoreType.DMA,
            pltpu.SemaphoreType.DMA,
            pltpu.SemaphoreType.DMA,
        ],
        compiler_params=pltpu.CompilerParams(use_tc_tiling_on_sc=False,
                                             needs_layout_passes=False),
    )
    return f(table, feats, p)


def kernel(feats, p, table):
    return _sve(table, feats, p)


# speedup vs baseline: 1.1343x; 1.1343x over previous
import jax
import jax.numpy as jnp
from jax import lax
from jax.experimental import pallas as pl
from jax.experimental.pallas import tpu as pltpu
from jax.experimental.pallas import tpu_sc as plsc

NUM_KEYS = 1000000
EMBED_DIM = 32
N_VOX = 262144

NC = 2
NS = 16
L = 16
NW = NC * NS
VPW = N_VOX // NW
C = 128
ROWS = C * 8
IROWS = ROWS // 128
OROWS = C * EMBED_DIM // 128
NCHUNK = VPW // C
FEAT_ROWS = N_VOX * 8 // 128
P_ROWS = N_VOX // 128


def _body(table_hbm, feats_hbm, p_hbm, out_hbm, idx_v, rows_v, p_v, out_v,
          gsem0, gsem1, ssem0, ssem1, osem0, osem1):
    cid = lax.axis_index("c")
    sid = lax.axis_index("s")
    wid = sid * NC + cid
    gsems = (gsem0, gsem1)
    ssems = (ssem0, ssem1)
    osems = (osem0, osem1)

    def stage_idx(g, b):
        vbase = wid * VPW + g * C
        irow = pl.multiple_of(vbase // 16, 8)
        pltpu.async_copy(feats_hbm.at[pl.ds(irow, IROWS)], idx_v.at[b],
                         ssems[b])

    def stage_p(g, b):
        vbase = wid * VPW + g * C
        prow = vbase // 128
        for d in range(3):
            pltpu.async_copy(
                p_hbm.at[pl.ds(d * P_ROWS + prow, 1)],
                p_v.at[b].at[pl.ds(d, 1)], ssems[b])

    def wait_stage(b):
        pltpu.make_async_copy(feats_hbm.at[pl.ds(0, IROWS)], idx_v.at[b],
                              ssems[b]).wait()
        for d in range(3):
            pltpu.make_async_copy(p_hbm.at[pl.ds(0, 1)],
                                  p_v.at[b].at[pl.ds(d, 1)],
                                  ssems[b]).wait()

    def fire(g, b):
        for j in range(IROWS):
            pltpu.async_copy(table_hbm.at[idx_v.at[b].at[j]],
                             rows_v.at[b].at[pl.ds(j * 128, 128)], gsems[b])

    def drain_gathers(b):
        pltpu.make_async_copy(table_hbm.at[pl.ds(0, ROWS)],
                              rows_v.at[b], gsems[b]).wait()

    def flush_out(g, b):
        obase = (wid * VPW + g * C) * EMBED_DIM // 128
        pltpu.async_copy(out_v.at[b], out_hbm.at[pl.ds(obase, OROWS)],
                         osems[b])

    def wait_out(b):
        pltpu.make_async_copy(out_hbm.at[pl.ds(0, OROWS)], out_v.at[b],
                              osems[b]).wait()

    def compute(g, b):
        def group(i, c2):
            vb = i * L
            px = p_v[b, 0, pl.ds(vb, L)]
            py = p_v[b, 1, pl.ds(vb, L)]
            pz = p_v[b, 2, pl.ds(vb, L)]
            qx = 1.0 - px
            qy = 1.0 - py
            qz = 1.0 - pz
            w = []
            for j in range(8):
                wx = px if (j >> 2) & 1 else qx
                wy = py if (j >> 1) & 1 else qy
                wz = pz if j & 1 else qz
                w.append(wx * wy * wz)
            for lane in range(L):
                rbase = (vb + lane) * 8
                for h in range(2):
                    t = [w[j][lane] * rows_v[b, rbase + j, pl.ds(h * L, L)]
                         for j in range(8)]
                    acc = ((t[0] + t[1]) + (t[2] + t[3])) + \
                          ((t[4] + t[5]) + (t[6] + t[7]))
                    s = lane * EMBED_DIM + h * L
                    out_v[b, 4 * i + s // 128, pl.ds(s % 128, L)] = acc
            return c2

        lax.fori_loop(0, C // L, group, 0)

    stage_idx(0, 0)
    stage_p(0, 0)
    wait_stage(0)
    fire(0, 0)
    stage_idx(1, 1)
    stage_p(1, 1)

    def pair(t, carry):
        for bb in range(2):
            g = 2 * t + bb
            b, nb = bb, 1 - bb
            drain_gathers(b)

            @pl.when(g + 1 < NCHUNK)
            def _():
                wait_stage(nb)
                fire(g + 1, nb)

            @pl.when(g + 2 < NCHUNK)
            def _():
                stage_idx(g + 2, b)

            @pl.when(g >= 2)
            def _():
                wait_out(b)

            compute(g, b)

            @pl.when(g + 2 < NCHUNK)
            def _():
                stage_p(g + 2, b)

            flush_out(g, b)
        return carry

    lax.fori_loop(0, NCHUNK // 2, pair, 0)
    wait_out(0)
    wait_out(1)


@jax.jit
def _sve(table, feats2d, p_rows):
    mesh = plsc.VectorSubcoreMesh(core_axis_name="c", subcore_axis_name="s",
                                  num_cores=NC, num_subcores=NS)
    f = pl.kernel(
        _body,
        out_type=jax.ShapeDtypeStruct((N_VOX * EMBED_DIM // 128, 128),
                                      jnp.float32),
        mesh=mesh,
        scratch_types=[
            pltpu.VMEM((2, IROWS, 128), jnp.int32),
            pltpu.VMEM((2, ROWS, EMBED_DIM), jnp.float32),
            pltpu.VMEM((2, 3, 128), jnp.float32),
            pltpu.VMEM((2, OROWS, 128), jnp.float32),
            pltpu.SemaphoreType.DMA,
            pltpu.SemaphoreType.DMA,
            pltpu.SemaphoreType.DMA,
            pltpu.SemaphoreType.DMA,
            pltpu.SemaphoreType.DMA,
            pltpu.SemaphoreType.DMA,
        ],
        compiler_params=pltpu.CompilerParams(use_tc_tiling_on_sc=False),
    )
    return f(table, feats2d, p_rows)


def kernel(feats, p, table):
    feats2d = feats.reshape(FEAT_ROWS, 128)
    p_rows = p.T.reshape(3 * P_ROWS, 128)
    return _sve(table, feats2d, p_rows).reshape(N_VOX, EMBED_DIM)


# speedup vs baseline: 1.1372x; 1.0025x over previous
import jax
import jax.numpy as jnp
from jax import lax
from jax.experimental import pallas as pl
from jax.experimental.pallas import tpu as pltpu
from jax.experimental.pallas import tpu_sc as plsc

NUM_KEYS = 1000000
EMBED_DIM = 32
N_VOX = 262144

NC = 2
NS = 16
L = 16
NW = NC * NS
VPW = N_VOX // NW
C = 128
ROWS = C * 8
IROWS = ROWS // 128
OROWS = C * EMBED_DIM // 128
NCHUNK = VPW // C
FEAT_ROWS = N_VOX * 8 // 128
P_ROWS = N_VOX // 128


def _body(table_hbm, feats_hbm, p_hbm, out_hbm, idx_v, rows_v, p_v, out_v,
          gsem0, gsem1, ssem0, ssem1, osem0, osem1):
    cid = lax.axis_index("c")
    sid = lax.axis_index("s")
    wid = sid * NC + cid
    gsems = (gsem0, gsem1)
    ssems = (ssem0, ssem1)
    osems = (osem0, osem1)

    def stage_idx(g, b):
        vbase = wid * VPW + g * C
        irow = pl.multiple_of(vbase // 16, 8)
        pltpu.async_copy(feats_hbm.at[pl.ds(irow, IROWS)], idx_v.at[b],
                         ssems[b])

    def stage_p(g, b):
        vbase = wid * VPW + g * C
        for d in range(3):
            pltpu.async_copy(
                p_hbm.at[pl.ds(d, 1), pl.ds(vbase, C)],
                p_v.at[b].at[pl.ds(d, 1)], ssems[b])

    def wait_stage(b):
        pltpu.make_async_copy(feats_hbm.at[pl.ds(0, IROWS)], idx_v.at[b],
                              ssems[b]).wait()
        for d in range(3):
            pltpu.make_async_copy(p_hbm.at[pl.ds(0, 1), pl.ds(0, C)],
                                  p_v.at[b].at[pl.ds(d, 1)],
                                  ssems[b]).wait()

    def fire(g, b):
        for j in range(IROWS):
            pltpu.async_copy(table_hbm.at[idx_v.at[b].at[j]],
                             rows_v.at[b].at[pl.ds(j * 128, 128)], gsems[b])

    def drain_gathers(b):
        pltpu.make_async_copy(table_hbm.at[pl.ds(0, ROWS)],
                              rows_v.at[b], gsems[b]).wait()

    def flush_out(g, b):
        obase = (wid * VPW + g * C) * EMBED_DIM // 128
        pltpu.async_copy(out_v.at[b], out_hbm.at[pl.ds(obase, OROWS)],
                         osems[b])

    def wait_out(b):
        pltpu.make_async_copy(out_hbm.at[pl.ds(0, OROWS)], out_v.at[b],
                              osems[b]).wait()

    def compute(g, b):
        def group(i, c2):
            vb = i * L
            px = p_v[b, 0, pl.ds(vb, L)]
            py = p_v[b, 1, pl.ds(vb, L)]
            pz = p_v[b, 2, pl.ds(vb, L)]
            qx = 1.0 - px
            qy = 1.0 - py
            qz = 1.0 - pz
            w = []
            for j in range(8):
                wx = px if (j >> 2) & 1 else qx
                wy = py if (j >> 1) & 1 else qy
                wz = pz if j & 1 else qz
                w.append(wx * wy * wz)
            for lane in range(L):
                rbase = (vb + lane) * 8
                for h in range(2):
                    t = [w[j][lane] * rows_v[b, rbase + j, pl.ds(h * L, L)]
                         for j in range(8)]
                    acc = ((t[0] + t[1]) + (t[2] + t[3])) + \
                          ((t[4] + t[5]) + (t[6] + t[7]))
                    s = lane * EMBED_DIM + h * L
                    out_v[b, 4 * i + s // 128, pl.ds(s % 128, L)] = acc
            return c2

        lax.fori_loop(0, C // L, group, 0)

    stage_idx(0, 0)
    stage_p(0, 0)
    wait_stage(0)
    fire(0, 0)
    stage_idx(1, 1)
    stage_p(1, 1)

    def pair(t, carry):
        for bb in range(2):
            g = 2 * t + bb
            b, nb = bb, 1 - bb
            drain_gathers(b)

            @pl.when(g + 1 < NCHUNK)
            def _():
                wait_stage(nb)
                fire(g + 1, nb)

            @pl.when(g + 2 < NCHUNK)
            def _():
                stage_idx(g + 2, b)

            @pl.when(g >= 2)
            def _():
                wait_out(b)

            compute(g, b)

            @pl.when(g + 2 < NCHUNK)
            def _():
                stage_p(g + 2, b)

            flush_out(g, b)
        return carry

    lax.fori_loop(0, NCHUNK // 2, pair, 0)
    wait_out(0)
    wait_out(1)


@jax.jit
def _sve(table, feats2d, p_rows):
    mesh = plsc.VectorSubcoreMesh(core_axis_name="c", subcore_axis_name="s",
                                  num_cores=NC, num_subcores=NS)
    f = pl.kernel(
        _body,
        out_type=jax.ShapeDtypeStruct((N_VOX * EMBED_DIM // 128, 128),
                                      jnp.float32),
        mesh=mesh,
        scratch_types=[
            pltpu.VMEM((2, IROWS, 128), jnp.int32),
            pltpu.VMEM((2, ROWS, EMBED_DIM), jnp.float32),
            pltpu.VMEM((2, 3, 128), jnp.float32),
            pltpu.VMEM((2, OROWS, 128), jnp.float32),
            pltpu.SemaphoreType.DMA,
            pltpu.SemaphoreType.DMA,
            pltpu.SemaphoreType.DMA,
            pltpu.SemaphoreType.DMA,
            pltpu.SemaphoreType.DMA,
            pltpu.SemaphoreType.DMA,
        ],
        compiler_params=pltpu.CompilerParams(use_tc_tiling_on_sc=False),
    )
    return f(table, feats2d, p_rows)


def kernel(feats, p, table):
    feats2d = feats.reshape(FEAT_ROWS, 128)
    return _sve(table, feats2d, p.T).reshape(N_VOX, EMBED_DIM)


# speedup vs baseline: 1.2533x; 1.1021x over previous
import jax
import jax.numpy as jnp
from jax import lax
from jax.experimental import pallas as pl
from jax.experimental.pallas import tpu as pltpu
from jax.experimental.pallas import tpu_sc as plsc

NUM_KEYS = 1000000
EMBED_DIM = 32
N_VOX = 262144

NC = 2
NS = 16
L = 16
NW = NC * NS
VPW = N_VOX // NW
C = 128
ROWS = C * 8
IROWS = ROWS // 128
OROWS = C * EMBED_DIM // 128
NCHUNK = VPW // C
FEAT_ROWS = N_VOX * 8 // 128
P_ROWS = N_VOX // 128


def _body(table_hbm, feats_hbm, p_hbm, out_hbm, idx_v, idx1d_v, rows_v,
          p_v, out_v, gsem0, gsem1, ssem0, ssem1, osem0, osem1):
    cid = lax.axis_index("c")
    sid = lax.axis_index("s")
    wid = sid * NC + cid
    gsems = (gsem0, gsem1)
    ssems = (ssem0, ssem1)
    osems = (osem0, osem1)

    def stage_idx(g, b):
        vbase = wid * VPW + g * C
        irow = pl.multiple_of(vbase // 16, 8)
        pltpu.async_copy(feats_hbm.at[pl.ds(irow, IROWS)], idx_v.at[b],
                         ssems[b])

    def stage_p(g, b):
        vbase = wid * VPW + g * C
        for d in range(3):
            pltpu.async_copy(
                p_hbm.at[pl.ds(d, 1), pl.ds(vbase, C)],
                p_v.at[b].at[pl.ds(d, 1)], ssems[b])

    def wait_stage(b):
        pltpu.make_async_copy(feats_hbm.at[pl.ds(0, IROWS)], idx_v.at[b],
                              ssems[b]).wait()
        for d in range(3):
            pltpu.make_async_copy(p_hbm.at[pl.ds(0, 1), pl.ds(0, C)],
                                  p_v.at[b].at[pl.ds(d, 1)],
                                  ssems[b]).wait()

    def transpose_idx(b):
        def tr(i, c2):
            v8 = (lax.iota(jnp.int32, L) + i * L) * 8
            for j in range(8):
                f = v8 + j
                vv = plsc.load_gather(
                    idx_v.at[b],
                    [lax.shift_right_logical(f, 7),
                     lax.bitwise_and(f, 127)])
                idx1d_v[b, j, pl.ds(i * L, L)] = vv
            return c2

        lax.fori_loop(0, C // L, tr, 0)

    def fire(g, b):
        for j in range(8):
            pltpu.async_copy(table_hbm.at[idx1d_v.at[b].at[j]],
                             rows_v.at[b].at[pl.ds(j * C, C)], gsems[b])

    def drain_gathers(b):
        pltpu.make_async_copy(table_hbm.at[pl.ds(0, ROWS)],
                              rows_v.at[b], gsems[b]).wait()

    def flush_out(g, b):
        obase = (wid * VPW + g * C) * EMBED_DIM // 128
        pltpu.async_copy(out_v.at[b], out_hbm.at[pl.ds(obase, OROWS)],
                         osems[b])

    def wait_out(b):
        pltpu.make_async_copy(out_hbm.at[pl.ds(0, OROWS)], out_v.at[b],
                              osems[b]).wait()

    def compute(g, b):
        def group(i, c2):
            vb = i * L
            px = p_v[b, 0, pl.ds(vb, L)]
            py = p_v[b, 1, pl.ds(vb, L)]
            pz = p_v[b, 2, pl.ds(vb, L)]
            qx = 1.0 - px
            qy = 1.0 - py
            qz = 1.0 - pz
            w = []
            for j in range(8):
                wx = px if (j >> 2) & 1 else qx
                wy = py if (j >> 1) & 1 else qy
                wz = pz if j & 1 else qz
                w.append(wx * wy * wz)
            for lane in range(L):
                rbase = vb + lane
                for h in range(2):
                    t = [w[j][lane] *
                         rows_v[b, j * C + rbase, pl.ds(h * L, L)]
                         for j in range(8)]
                    acc = ((t[0] + t[1]) + (t[2] + t[3])) + \
                          ((t[4] + t[5]) + (t[6] + t[7]))
                    s = lane * EMBED_DIM + h * L
                    out_v[b, 4 * i + s // 128, pl.ds(s % 128, L)] = acc
            return c2

        lax.fori_loop(0, C // L, group, 0)

    stage_idx(0, 0)
    stage_p(0, 0)
    wait_stage(0)
    transpose_idx(0)
    fire(0, 0)
    stage_idx(1, 1)
    stage_p(1, 1)

    def pair(t, carry):
        for bb in range(2):
            g = 2 * t + bb
            b, nb = bb, 1 - bb
            drain_gathers(b)

            @pl.when(g + 1 < NCHUNK)
            def _():
                wait_stage(nb)
                transpose_idx(nb)
                fire(g + 1, nb)

            @pl.when(g + 2 < NCHUNK)
            def _():
                stage_idx(g + 2, b)

            @pl.when(g >= 2)
            def _():
                wait_out(b)

            compute(g, b)

            @pl.when(g + 2 < NCHUNK)
            def _():
                stage_p(g + 2, b)

            flush_out(g, b)
        return carry

    lax.fori_loop(0, NCHUNK // 2, pair, 0)
    wait_out(0)
    wait_out(1)


@jax.jit
def _sve(table, feats2d, p_rows):
    mesh = plsc.VectorSubcoreMesh(core_axis_name="c", subcore_axis_name="s",
                                  num_cores=NC, num_subcores=NS)
    f = pl.kernel(
        _body,
        out_type=jax.ShapeDtypeStruct((N_VOX * EMBED_DIM // 128, 128),
                                      jnp.float32),
        mesh=mesh,
        scratch_types=[
            pltpu.VMEM((2, IROWS, 128), jnp.int32),
            pltpu.VMEM((2, 8, C), jnp.int32),
            pltpu.VMEM((2, ROWS, EMBED_DIM), jnp.float32),
            pltpu.VMEM((2, 3, 128), jnp.float32),
            pltpu.VMEM((2, OROWS, 128), jnp.float32),
            pltpu.SemaphoreType.DMA,
            pltpu.SemaphoreType.DMA,
            pltpu.SemaphoreType.DMA,
            pltpu.SemaphoreType.DMA,
            pltpu.SemaphoreType.DMA,
            pltpu.SemaphoreType.DMA,
        ],
        compiler_params=pltpu.CompilerParams(use_tc_tiling_on_sc=False,
                                             needs_layout_passes=False),
    )
    return f(table, feats2d, p_rows)


def kernel(feats, p, table):
    feats2d = feats.reshape(FEAT_ROWS, 128)
    return _sve(table, feats2d, p.T).reshape(N_VOX, EMBED_DIM)
